# baseline (speedup 1.0000x reference)
import functools

import jax
import jax.numpy as jnp
from jax import lax
from jax.experimental import pallas as pl
from jax.experimental.pallas import tpu as pltpu
from jax.experimental.pallas import tpu_sc as plsc

_G = 128
_W = 128


@jax.jit
def _encode(category, operator_class, cat_emb_pad, op_emb_pad):
    info = plsc.get_sparse_core_info()
    nw = info.num_cores * info.num_subcores
    n = category.shape[0]
    n_full = n // _G
    tail = n - n_full * _G
    full_per_w_lo = n_full // nw
    n_extra = n_full - full_per_w_lo * nw
    tail_w = n_full % nw

    mesh = plsc.VectorSubcoreMesh(core_axis_name="c", subcore_axis_name="s")

    @functools.partial(
        pl.kernel,
        mesh=mesh,
        compiler_params=pltpu.CompilerParams(needs_layout_passes=False),
        out_type=(jax.ShapeDtypeStruct((n, _W), jnp.float32),
                  jax.ShapeDtypeStruct((n, _W), jnp.float32)),
        scratch_types=[
            pltpu.VMEM((_G,), jnp.int32),
            pltpu.VMEM((_G,), jnp.int32),
            pltpu.VMEM((_G,), jnp.int32),
            pltpu.VMEM((_G,), jnp.int32),
            pltpu.VMEM((_G, _W), jnp.float32),
            pltpu.VMEM((_G, _W), jnp.float32),
            pltpu.VMEM((_G, _W), jnp.float32),
            pltpu.VMEM((_G, _W), jnp.float32),
            pltpu.SemaphoreType.DMA,
            pltpu.SemaphoreType.DMA,
            pltpu.SemaphoreType.DMA,
            pltpu.SemaphoreType.DMA,
            pltpu.SemaphoreType.DMA,
            pltpu.SemaphoreType.DMA,
            pltpu.SemaphoreType.DMA,
            pltpu.SemaphoreType.DMA,
        ],
    )
    def k(cat_idx_hbm, op_idx_hbm, cat_tab_hbm, op_tab_hbm,
          cat_g_hbm, op_g_hbm,
          idxc_a, idxo_a, idxc_b, idxo_b, catbuf_a, catbuf_b,
          opbuf_a, opbuf_b,
          isem_a, isem_b, gsem_a, gsem_b, wc_a, wc_b, wo_a, wo_b):
        wid = lax.axis_index("s") * info.num_cores + lax.axis_index("c")
        n_full_w = full_per_w_lo + jnp.where(wid < n_extra, 1, 0)

        def rowof(t):
            return (wid + t * nw) * _G

        def start_a(t, idxc_r, idxo_r, isem_r):
            row0 = rowof(t)
            pltpu.async_copy(cat_idx_hbm.at[pl.ds(row0, _G)], idxc_r, isem_r)
            pltpu.async_copy(op_idx_hbm.at[pl.ds(row0, _G)], idxo_r, isem_r)

        def start_b(t, idxc_r, idxo_r, catbuf_r, opbuf_r, gsem, isem_r):
            row0 = rowof(t)
            pltpu.make_async_copy(
                cat_idx_hbm.at[pl.ds(row0, _G)], idxc_r, isem_r).wait()
            pltpu.make_async_copy(
                op_idx_hbm.at[pl.ds(row0, _G)], idxo_r, isem_r).wait()
            pltpu.async_copy(cat_tab_hbm.at[idxc_r], catbuf_r, gsem)
            pltpu.async_copy(op_tab_hbm.at[idxo_r], opbuf_r, gsem)

        def phase(t, idxc_r, idxo_r, idxc_o, idxo_o, catbuf_r, catbuf_o,
                  opbuf_r, opbuf_o, gsem, gsem_o, wc, wc_o, wo, wo_o,
                  isem_r, isem_o):
            pltpu.make_async_copy(
                cat_tab_hbm.at[idxc_r], catbuf_r, gsem).wait()
            pltpu.make_async_copy(
                op_tab_hbm.at[idxo_r], opbuf_r, gsem).wait()

            @pl.when(t + 2 < n_full_w)
            def _():
                start_a(t + 2, idxc_r, idxo_r, isem_r)

            row0 = rowof(t)
            pltpu.async_copy(catbuf_r, cat_g_hbm.at[pl.ds(row0, _G)], wc)
            pltpu.async_copy(opbuf_r, op_g_hbm.at[pl.ds(row0, _G)], wo)

            @pl.when(t >= 1)
            def _():
                pltpu.make_async_copy(
                    catbuf_o, cat_g_hbm.at[pl.ds(0, _G)], wc_o).wait()
                pltpu.make_async_copy(
                    opbuf_o, op_g_hbm.at[pl.ds(0, _G)], wo_o).wait()

            @pl.when(t + 1 < n_full_w)
            def _():
                start_b(t + 1, idxc_o, idxo_o, catbuf_o, opbuf_o, gsem_o,
                        isem_o)

        start_a(0, idxc_a, idxo_a, isem_a)

        @pl.when(n_full_w >= 2)
        def _():
            start_a(1, idxc_b, idxo_b, isem_b)
        start_b(0, idxc_a, idxo_a, catbuf_a, opbuf_a, gsem_a, isem_a)

        def body(t, carry):
            @pl.when((t & 1) == 0)
            def _():
                phase(t, idxc_a, idxo_a, idxc_b, idxo_b, catbuf_a, catbuf_b,
                      opbuf_a, opbuf_b, gsem_a, gsem_b, wc_a, wc_b,
                      wo_a, wo_b, isem_a, isem_b)

            @pl.when((t & 1) == 1)
            def _():
                phase(t, idxc_b, idxo_b, idxc_a, idxo_a, catbuf_b, catbuf_a,
                      opbuf_b, opbuf_a, gsem_b, gsem_a, wc_b, wc_a,
                      wo_b, wo_a, isem_b, isem_a)
            return carry

        lax.fori_loop(0, n_full_w, body, 0)

        @pl.when(wid < n_extra)
        def _():
            pltpu.make_async_copy(
                catbuf_a, cat_g_hbm.at[pl.ds(0, _G)], wc_a).wait()
            pltpu.make_async_copy(
                opbuf_a, op_g_hbm.at[pl.ds(0, _G)], wo_a).wait()

        @pl.when(jnp.logical_not(wid < n_extra))
        def _():
            pltpu.make_async_copy(
                catbuf_b, cat_g_hbm.at[pl.ds(0, _G)], wc_b).wait()
            pltpu.make_async_copy(
                opbuf_b, op_g_hbm.at[pl.ds(0, _G)], wo_b).wait()

        if tail:
            @pl.when(wid == tail_w)
            def _():
                row0 = n_full * _G
                pltpu.sync_copy(cat_idx_hbm.at[pl.ds(row0, tail)],
                                idxc_a.at[pl.ds(0, tail)])
                pltpu.sync_copy(op_idx_hbm.at[pl.ds(row0, tail)],
                                idxo_a.at[pl.ds(0, tail)])
                a = pltpu.async_copy(
                    cat_tab_hbm.at[idxc_a.at[pl.ds(0, tail)]],
                    catbuf_a.at[pl.ds(0, tail)], gsem_a)
                b = pltpu.async_copy(
                    op_tab_hbm.at[idxo_a.at[pl.ds(0, tail)]],
                    opbuf_a.at[pl.ds(0, tail)], gsem_a)
                a.wait()
                b.wait()
                pltpu.sync_copy(catbuf_a.at[pl.ds(0, tail)],
                                cat_g_hbm.at[pl.ds(row0, tail)])
                pltpu.sync_copy(opbuf_a.at[pl.ds(0, tail)],
                                op_g_hbm.at[pl.ds(row0, tail)])

    return k(category, operator_class, cat_emb_pad, op_emb_pad)


def kernel(category, operator_class, rest_features, cat_emb, op_emb):
    d_cat = cat_emb.shape[1]
    d_op = op_emb.shape[1]
    cat_emb_pad = jnp.concatenate(
        [cat_emb, jnp.zeros((cat_emb.shape[0], _W - d_cat), jnp.float32)], 1)
    op_emb_pad = jnp.concatenate(
        [op_emb, jnp.zeros((op_emb.shape[0], _W - d_op), jnp.float32)], 1)
    cat_g, op_g = _encode(category.astype(jnp.int32),
                          operator_class.astype(jnp.int32),
                          cat_emb_pad, op_emb_pad)
    out_t = jnp.concatenate(
        [cat_g[:, :d_cat].T, op_g[:, :d_op].T, rest_features.T], axis=0)
    return out_t.T


# speedup vs baseline: 1.2344x; 1.2344x over previous
import functools

import jax
import jax.numpy as jnp
from jax import lax
from jax.experimental import pallas as pl
from jax.experimental.pallas import tpu as pltpu
from jax.experimental.pallas import tpu_sc as plsc

_G = 128
_W = 128


@jax.jit
def _encode(category, operator_class, cat_emb_pad, op_emb_pad):
    info = plsc.get_sparse_core_info()
    nw = info.num_cores * info.num_subcores
    n = category.shape[0]
    n_full = n // _G
    tail = n - n_full * _G
    full_per_w_lo = n_full // nw
    n_extra = n_full - full_per_w_lo * nw
    tail_w = n_full % nw

    mesh = plsc.VectorSubcoreMesh(core_axis_name="c", subcore_axis_name="s")

    @functools.partial(
        pl.kernel,
        mesh=mesh,
        compiler_params=pltpu.CompilerParams(needs_layout_passes=False),
        out_type=(jax.ShapeDtypeStruct((n, _W), jnp.float32),
                  jax.ShapeDtypeStruct((n, _W), jnp.float32)),
        scratch_types=[
            pltpu.VMEM((_G,), jnp.int32),
            pltpu.VMEM((_G,), jnp.int32),
            pltpu.VMEM((_G,), jnp.int32),
            pltpu.VMEM((_G,), jnp.int32),
            pltpu.VMEM((_G, _W), jnp.float32),
            pltpu.VMEM((_G, _W), jnp.float32),
            pltpu.VMEM((_G, _W), jnp.float32),
            pltpu.VMEM((_G, _W), jnp.float32),
            pltpu.SemaphoreType.DMA,
            pltpu.SemaphoreType.DMA,
            pltpu.SemaphoreType.DMA,
            pltpu.SemaphoreType.DMA,
            pltpu.SemaphoreType.DMA,
            pltpu.SemaphoreType.DMA,
            pltpu.SemaphoreType.DMA,
            pltpu.SemaphoreType.DMA,
        ],
    )
    def k(cat_idx_hbm, op_idx_hbm, cat_tab_hbm, op_tab_hbm,
          cat_g_hbm, op_g_hbm,
          idxc_a, idxo_a, idxc_b, idxo_b, catbuf_a, catbuf_b,
          opbuf_a, opbuf_b,
          isem_a, isem_b, gsem_a, gsem_b, wc_a, wc_b, wo_a, wo_b):
        wid = lax.axis_index("s") * info.num_cores + lax.axis_index("c")
        n_full_w = full_per_w_lo + jnp.where(wid < n_extra, 1, 0)

        def rowof(t):
            return (wid + t * nw) * _G

        def start_a(t, idxc_r, idxo_r, isem_r):
            row0 = rowof(t)
            pltpu.async_copy(cat_idx_hbm.at[pl.ds(row0, _G)], idxc_r, isem_r)
            pltpu.async_copy(op_idx_hbm.at[pl.ds(row0, _G)], idxo_r, isem_r)

        def start_b(t, idxc_r, idxo_r, catbuf_r, opbuf_r, gsem, isem_r):
            row0 = rowof(t)
            pltpu.make_async_copy(
                cat_idx_hbm.at[pl.ds(row0, _G)], idxc_r, isem_r).wait()
            pltpu.make_async_copy(
                op_idx_hbm.at[pl.ds(row0, _G)], idxo_r, isem_r).wait()
            pltpu.async_copy(cat_tab_hbm.at[idxc_r], catbuf_r, gsem)
            pltpu.async_copy(op_tab_hbm.at[idxo_r], opbuf_r, gsem)

        def phase(t, idxc_r, idxo_r, idxc_o, idxo_o, catbuf_r, catbuf_o,
                  opbuf_r, opbuf_o, gsem, gsem_o, wc, wc_o, wo, wo_o,
                  isem_r, isem_o):
            pltpu.make_async_copy(
                cat_tab_hbm.at[idxc_r], catbuf_r, gsem).wait()
            pltpu.make_async_copy(
                op_tab_hbm.at[idxo_r], opbuf_r, gsem).wait()

            @pl.when(t + 2 < n_full_w)
            def _():
                start_a(t + 2, idxc_r, idxo_r, isem_r)

            row0 = rowof(t)
            pltpu.async_copy(catbuf_r, cat_g_hbm.at[pl.ds(row0, _G)], wc)
            pltpu.async_copy(opbuf_r, op_g_hbm.at[pl.ds(row0, _G)], wo)

            @pl.when(t >= 1)
            def _():
                pltpu.make_async_copy(
                    catbuf_o, cat_g_hbm.at[pl.ds(0, _G)], wc_o).wait()
                pltpu.make_async_copy(
                    opbuf_o, op_g_hbm.at[pl.ds(0, _G)], wo_o).wait()

            @pl.when(t + 1 < n_full_w)
            def _():
                start_b(t + 1, idxc_o, idxo_o, catbuf_o, opbuf_o, gsem_o,
                        isem_o)

        start_a(0, idxc_a, idxo_a, isem_a)

        @pl.when(n_full_w >= 2)
        def _():
            start_a(1, idxc_b, idxo_b, isem_b)
        start_b(0, idxc_a, idxo_a, catbuf_a, opbuf_a, gsem_a, isem_a)

        def body(t, carry):
            @pl.when((t & 1) == 0)
            def _():
                phase(t, idxc_a, idxo_a, idxc_b, idxo_b, catbuf_a, catbuf_b,
                      opbuf_a, opbuf_b, gsem_a, gsem_b, wc_a, wc_b,
                      wo_a, wo_b, isem_a, isem_b)

            @pl.when((t & 1) == 1)
            def _():
                phase(t, idxc_b, idxo_b, idxc_a, idxo_a, catbuf_b, catbuf_a,
                      opbuf_b, opbuf_a, gsem_b, gsem_a, wc_b, wc_a,
                      wo_b, wo_a, isem_b, isem_a)
            return carry

        lax.fori_loop(0, n_full_w, body, 0)

        @pl.when(wid < n_extra)
        def _():
            pltpu.make_async_copy(
                catbuf_a, cat_g_hbm.at[pl.ds(0, _G)], wc_a).wait()
            pltpu.make_async_copy(
                opbuf_a, op_g_hbm.at[pl.ds(0, _G)], wo_a).wait()

        @pl.when(jnp.logical_not(wid < n_extra))
        def _():
            pltpu.make_async_copy(
                catbuf_b, cat_g_hbm.at[pl.ds(0, _G)], wc_b).wait()
            pltpu.make_async_copy(
                opbuf_b, op_g_hbm.at[pl.ds(0, _G)], wo_b).wait()

        if tail:
            @pl.when(wid == tail_w)
            def _():
                row0 = n_full * _G
                pltpu.sync_copy(cat_idx_hbm.at[pl.ds(row0, tail)],
                                idxc_a.at[pl.ds(0, tail)])
                pltpu.sync_copy(op_idx_hbm.at[pl.ds(row0, tail)],
                                idxo_a.at[pl.ds(0, tail)])
                a = pltpu.async_copy(
                    cat_tab_hbm.at[idxc_a.at[pl.ds(0, tail)]],
                    catbuf_a.at[pl.ds(0, tail)], gsem_a)
                b = pltpu.async_copy(
                    op_tab_hbm.at[idxo_a.at[pl.ds(0, tail)]],
                    opbuf_a.at[pl.ds(0, tail)], gsem_a)
                a.wait()
                b.wait()
                pltpu.sync_copy(catbuf_a.at[pl.ds(0, tail)],
                                cat_g_hbm.at[pl.ds(row0, tail)])
                pltpu.sync_copy(opbuf_a.at[pl.ds(0, tail)],
                                op_g_hbm.at[pl.ds(row0, tail)])

    return k(category, operator_class, cat_emb_pad, op_emb_pad)


def kernel(category, operator_class, rest_features, cat_emb, op_emb):
    d_cat = cat_emb.shape[1]
    d_op = op_emb.shape[1]
    cat_emb_pad = jnp.concatenate(
        [cat_emb, jnp.zeros((cat_emb.shape[0], _W - d_cat), jnp.float32)], 1)
    op_emb_pad = jnp.concatenate(
        [op_emb, jnp.zeros((op_emb.shape[0], _W - d_op), jnp.float32)], 1)
    cat_g, op_g = _encode(category.astype(jnp.int32),
                          operator_class.astype(jnp.int32),
                          cat_emb_pad, op_emb_pad)
    return jnp.concatenate(
        [cat_g[:, :d_cat], op_g[:, :d_op], rest_features], axis=1)
